# Initial kernel scaffold; baseline (speedup 1.0000x reference)
import jax
import jax.numpy as jnp
from jax.experimental import pallas as pl


def kernel(x, groups, W, b):
    raise NotImplementedError("write your pallas kernel here")


# speedup vs baseline: 9.2393x; 9.2393x over previous
import jax
import jax.numpy as jnp
from jax import lax
from jax.experimental import pallas as pl
from jax.experimental.pallas import tpu as pltpu

B, S, D, G = 16, 4096, 256, 512


def _tc_body(x_ref, g_ref, w_ref, bias_ref, o_ref):
    ids = g_ref[0]
    gidx = lax.broadcasted_iota(jnp.int32, (G, S), 0)
    eq = ids == gidx
    m = jnp.where(eq, 1.0, 0.0)
    cnt = jnp.sum(m, axis=1, keepdims=True)
    seg = jax.lax.dot_general(
        m.astype(jnp.bfloat16), x_ref[0].astype(jnp.bfloat16),
        (((1,), (0,)), ((), ())),
        preferred_element_type=jnp.float32)
    mean = seg * (1.0 / jnp.maximum(cnt, 1.0))
    o_ref[0] = (
        jax.lax.dot_general(
            mean, w_ref[...], (((1,), (0,)), ((), ())),
            precision=jax.lax.Precision.HIGHEST,
            preferred_element_type=jnp.float32)
        + bias_ref[...]
    )


def kernel(x, groups, W, b):
    return pl.pallas_call(
        _tc_body,
        grid=(B,),
        in_specs=[
            pl.BlockSpec((1, S, D), lambda i: (i, 0, 0)),
            pl.BlockSpec((1, 1, S), lambda i: (i, 0, 0)),
            pl.BlockSpec((D, D), lambda i: (0, 0)),
            pl.BlockSpec((1, D), lambda i: (0, 0)),
        ],
        out_specs=pl.BlockSpec((1, G, D), lambda i: (i, 0, 0)),
        out_shape=jax.ShapeDtypeStruct((B, G, D), jnp.float32),
    )(x, groups.reshape(B, 1, S), W, b.reshape(1, D))
